# baseline (speedup 1.0000x reference)
import functools

import jax
import jax.numpy as jnp
from jax import lax
from jax.experimental import pallas as pl
from jax.experimental.pallas import tpu as pltpu
from jax.experimental.pallas import tpu_sc as plsc

_D = 128
_MX = 256
_MY = 256
_B = 4096
_L = 50
_N = _B * _L
_WIN = 160
_NWIN = _N // _WIN
_NROW = _N // _D
_LANES = 16


def _tc_prep(xr, yr, exh, eyh, e00, W, b):
    def body(xr_ref, yr_ref, exh_ref, eyh_ref, e00_ref, w_ref, b_ref,
             axb_ref, ay_ref, ix_ref, iy_ref):
        wm = w_ref[...]
        dn = (((1,), (1,)), ((), ()))
        axb_ref[...] = lax.dot_general(
            exh_ref[...], wm, dn, preferred_element_type=jnp.float32
        ) + b_ref[...]
        ay_ref[...] = lax.dot_general(
            eyh_ref[...] - e00_ref[...], wm, dn,
            preferred_element_type=jnp.float32)
        ix_ref[...] = (xr_ref[...] * (_MX - 1.0)).astype(jnp.int32) * _D
        iy_ref[...] = (yr_ref[...] * (_MY - 1.0)).astype(jnp.int32) * _D

    return pl.pallas_call(
        body,
        out_shape=(
            jax.ShapeDtypeStruct((_MX, _D), jnp.float32),
            jax.ShapeDtypeStruct((_MY, _D), jnp.float32),
            jax.ShapeDtypeStruct((_NROW, _D), jnp.int32),
            jax.ShapeDtypeStruct((_NROW, _D), jnp.int32),
        ),
    )(xr, yr, exh, eyh, e00, W, b)


def _sc_lookup(axbf, ayf, ix2, iy2):
    mesh = plsc.VectorSubcoreMesh(core_axis_name="core",
                                  subcore_axis_name="subcore")

    @functools.partial(
        pl.kernel,
        out_type=jax.ShapeDtypeStruct((_N * _D,), jnp.float32),
        mesh=mesh,
        scratch_types=[pltpu.VMEM((_MX * _D,), jnp.float32),
                       pltpu.VMEM((_MY * _D,), jnp.float32)],
        compiler_params=pltpu.CompilerParams(needs_layout_passes=False),
    )
    def kern(axb_hbm, ay_hbm, ix_hbm, iy_hbm, o_hbm, ta_vmem, tb_vmem):
        pltpu.sync_copy(axb_hbm, ta_vmem)
        pltpu.sync_copy(ay_hbm, tb_vmem)

        def body(ix_vmem, iy_vmem, o_vmem):
            @plsc.parallel_loop(0, _WIN // _LANES)
            def _grp(g):
                rx = ix_vmem[0, pl.ds(g * _LANES, _LANES)]
                ry = iy_vmem[0, pl.ds(g * _LANES, _LANES)]
                ob = (g * _LANES) * _D
                nk = _D // _LANES
                pend = None
                for l in range(_LANES):
                    ixp = rx[l]
                    iyp = ry[l]
                    a = []
                    bv = []
                    for k in range(nk):
                        a.append(ta_vmem[pl.ds(ixp + k * _LANES, _LANES)])
                        if pend is not None:
                            o_vmem[pl.ds(pend[1] + k * _LANES, _LANES)] = (
                                pend[0][k])
                        bv.append(tb_vmem[pl.ds(iyp + k * _LANES, _LANES)])
                    s = [jnp.maximum(a[k] + bv[k], 0.0) for k in range(nk)]
                    pend = (s, ob + l * _D)
                for k in range(nk):
                    o_vmem[pl.ds(pend[1] + k * _LANES, _LANES)] = pend[0][k]

        pltpu.emit_pipeline(
            body,
            grid=(_NWIN,),
            in_specs=[pl.BlockSpec((1, _WIN), lambda i: (i, 0)),
                      pl.BlockSpec((1, _WIN), lambda i: (i, 0))],
            out_specs=[pl.BlockSpec((_WIN * _D,), lambda i: (i,))],
            core_axis_name=("core", "subcore"),
            dimension_semantics=(pltpu.PARALLEL,),
        )(ix_hbm, iy_hbm, o_hbm)

    return kern(axbf, ayf, ix2, iy2)


def kernel(x, y, W, b, encoding):
    exh = encoding[:, 0, :]
    eyh = encoding[0, :, :]
    e00 = encoding[0:1, 0, :]
    xr = x.T.reshape(_NROW, _D)
    yr = y.T.reshape(_NROW, _D)
    axb, ay, ix, iy = _tc_prep(xr, yr, exh, eyh, e00, W, b.reshape(1, _D))
    out = _sc_lookup(axb.reshape(-1), ay.reshape(-1),
                     ix.reshape(_NWIN, _WIN), iy.reshape(_NWIN, _WIN))
    return out.reshape(_L, _B, _D).transpose(1, 0, 2)


# speedup vs baseline: 1.0628x; 1.0628x over previous
import functools

import jax
import jax.numpy as jnp
from jax import lax
from jax.experimental import pallas as pl
from jax.experimental.pallas import tpu as pltpu
from jax.experimental.pallas import tpu_sc as plsc

_D = 128
_MX = 256
_MY = 256
_B = 4096
_L = 50
_N = _B * _L
_WIN = 128
_NWIN = _N // _WIN
_NROW = _N // _D
_LANES = 16


def _tc_prep(xr, yr, exh, eyh, e00, W, b):
    def body(xr_ref, yr_ref, exh_ref, eyh_ref, e00_ref, w_ref, b_ref,
             axb_ref, ay_ref, ix_ref, iy_ref):
        wm = w_ref[...]
        dn = (((1,), (1,)), ((), ()))
        axb_ref[...] = lax.dot_general(
            exh_ref[...], wm, dn, preferred_element_type=jnp.float32
        ) + b_ref[...]
        ay_ref[...] = lax.dot_general(
            eyh_ref[...] - e00_ref[...], wm, dn,
            preferred_element_type=jnp.float32)
        ix_ref[...] = (xr_ref[...] * (_MX - 1.0)).astype(jnp.int32) * _D
        iy_ref[...] = (yr_ref[...] * (_MY - 1.0)).astype(jnp.int32) * _D

    return pl.pallas_call(
        body,
        out_shape=(
            jax.ShapeDtypeStruct((_MX, _D), jnp.float32),
            jax.ShapeDtypeStruct((_MY, _D), jnp.float32),
            jax.ShapeDtypeStruct((_NROW, _D), jnp.int32),
            jax.ShapeDtypeStruct((_NROW, _D), jnp.int32),
        ),
    )(xr, yr, exh, eyh, e00, W, b)


def _sc_lookup(axbf, ayf, ix2, iy2):
    mesh = plsc.VectorSubcoreMesh(core_axis_name="core",
                                  subcore_axis_name="subcore")

    @functools.partial(
        pl.kernel,
        out_type=jax.ShapeDtypeStruct((_N * _D,), jnp.float32),
        mesh=mesh,
        scratch_types=[pltpu.VMEM((_MX * _D,), jnp.float32),
                       pltpu.VMEM((_MY * _D,), jnp.float32)],
        compiler_params=pltpu.CompilerParams(needs_layout_passes=False),
    )
    def kern(axb_hbm, ay_hbm, ix_hbm, iy_hbm, o_hbm, ta_vmem, tb_vmem):
        pltpu.sync_copy(axb_hbm, ta_vmem)
        pltpu.sync_copy(ay_hbm, tb_vmem)

        def body(ix_vmem, iy_vmem, o_vmem):
            @plsc.parallel_loop(0, _WIN // _LANES)
            def _grp(g):
                rx = ix_vmem[0, pl.ds(g * _LANES, _LANES)]
                ry = iy_vmem[0, pl.ds(g * _LANES, _LANES)]
                ob = (g * _LANES) * _D
                nk = _D // _LANES
                pend = None
                for l in range(_LANES):
                    ixp = rx[l]
                    iyp = ry[l]
                    a = []
                    bv = []
                    for k in range(nk):
                        a.append(ta_vmem[pl.ds(ixp + k * _LANES, _LANES)])
                        if pend is not None:
                            o_vmem[pl.ds(pend[1] + k * _LANES, _LANES)] = (
                                pend[0][k])
                        bv.append(tb_vmem[pl.ds(iyp + k * _LANES, _LANES)])
                    s = [jnp.maximum(a[k] + bv[k], 0.0) for k in range(nk)]
                    pend = (s, ob + l * _D)
                for k in range(nk):
                    o_vmem[pl.ds(pend[1] + k * _LANES, _LANES)] = pend[0][k]

        pltpu.emit_pipeline(
            body,
            grid=(_NWIN,),
            in_specs=[pl.BlockSpec((1, _WIN), lambda i: (i, 0)),
                      pl.BlockSpec((1, _WIN), lambda i: (i, 0))],
            out_specs=[pl.BlockSpec((_WIN * _D,), lambda i: (i,))],
            core_axis_name=("core", "subcore"),
            dimension_semantics=(pltpu.PARALLEL,),
        )(ix_hbm, iy_hbm, o_hbm)

    return kern(axbf, ayf, ix2, iy2)


def kernel(x, y, W, b, encoding):
    exh = encoding[:, 0, :]
    eyh = encoding[0, :, :]
    e00 = encoding[0:1, 0, :]
    xr = x.T.reshape(_NROW, _D)
    yr = y.T.reshape(_NROW, _D)
    axb, ay, ix, iy = _tc_prep(xr, yr, exh, eyh, e00, W, b.reshape(1, _D))
    out = _sc_lookup(axb.reshape(-1), ay.reshape(-1), ix, iy)
    return out.reshape(_L, _B, _D).transpose(1, 0, 2)
